# baseline (speedup 1.0000x reference)
import functools

import jax
import jax.numpy as jnp
from jax import lax
from jax.experimental import pallas as pl
from jax.experimental.pallas import tpu as pltpu
from jax.experimental.pallas import tpu_sc as plsc

B, N, M = 16, 65536, 4096
NC, NS, L = 2, 16, 16
NW = NC * NS
G = (B * M) // NW
GW = 3 * G


def _make_sc_gather():
    mesh = plsc.VectorSubcoreMesh(core_axis_name="c", subcore_axis_name="s")

    @functools.partial(
        pl.kernel,
        mesh=mesh,
        compiler_params=pltpu.CompilerParams(
            use_tc_tiling_on_sc=False, needs_layout_passes=False),
        out_type=jax.ShapeDtypeStruct((B * M * 3,), jnp.float32),
        scratch_types=[
            pltpu.VMEM((G,), jnp.int32),
            pltpu.VMEM((GW,), jnp.int32),
            pltpu.VMEM((GW,), jnp.int32),
            pltpu.VMEM((GW,), jnp.float32),
            pltpu.SemaphoreType.DMA,
        ],
    )
    def sc_gather(xyz_hbm, idx_hbm, j3_hbm, out_hbm, idx_v, j3_v, ent_v,
                  words_v, sem):
        wid = lax.axis_index("s") * NC + lax.axis_index("c")
        base = wid * G
        boff3 = (base // M) * (3 * N)

        pltpu.sync_copy(idx_hbm.at[pl.ds(base, G)], idx_v)
        pltpu.sync_copy(j3_hbm, j3_v)

        lanes = lax.iota(jnp.int32, L)

        def build(i, carry):
            sl = pl.ds(i * L, L)
            jvec = j3_v[sl]
            g = plsc.load_gather(idx_v, [jvec])
            pvec = i * L + lanes
            ent_v[sl] = g * 3 + (pvec - jvec * 3) + boff3
            return carry

        lax.fori_loop(0, GW // L, build, 0)

        pltpu.async_copy(xyz_hbm.at[ent_v], words_v, sem).wait()
        pltpu.sync_copy(words_v, out_hbm.at[pl.ds(base * 3, GW)])

    return sc_gather


_sc_gather = _make_sc_gather()


def kernel(batch_sample_xyz, input):
    xyz_flat = batch_sample_xyz.reshape(B * N * 3)
    idx_flat = input.reshape(B * M).astype(jnp.int32)
    j3 = jnp.arange(GW, dtype=jnp.int32) // 3
    out = _sc_gather(xyz_flat, idx_flat, j3)
    return out.reshape(B, M, 3)


# speedup vs baseline: 115.1274x; 115.1274x over previous
import functools

import jax
import jax.numpy as jnp
from jax import lax
from jax.experimental import pallas as pl
from jax.experimental.pallas import tpu as pltpu
from jax.experimental.pallas import tpu_sc as plsc

B, N, M = 16, 65536, 4096
NC, NS, L = 2, 16, 16
NW = NC * NS
BH, BL = B // 8, 8
MH, ML = M // 128, 128
NH = N // 128
U = BH * MH
UW = BL * ML
PLANE = B * N
OPLANE = B * M


def _make_sc_gather():
    mesh = plsc.VectorSubcoreMesh(core_axis_name="c", subcore_axis_name="s")

    @functools.partial(
        pl.kernel,
        mesh=mesh,
        compiler_params=pltpu.CompilerParams(
            use_tc_tiling_on_sc=False, needs_layout_passes=False),
        out_type=jax.ShapeDtypeStruct((3 * B * M,), jnp.float32),
        scratch_types=[
            pltpu.VMEM((UW,), jnp.int32),
            pltpu.VMEM((3 * UW,), jnp.int32),
            pltpu.VMEM((3 * UW,), jnp.float32),
            pltpu.SemaphoreType.DMA,
        ],
    )
    def sc_gather(xyz_hbm, idx_hbm, out_hbm, idx_v, ent_v, words_v, sem):
        wid = lax.axis_index("s") * NC + lax.axis_index("c")

        for k in range(2):
            u = wid * 2 + k
            b_hi = u // MH
            m_hi = u % MH
            a0 = b_hi * (BL * N)

            pltpu.sync_copy(idx_hbm.at[pl.ds(u * UW, UW)], idx_v)

            def build(i, carry):
                g = idx_v[pl.ds(i * L, L)]
                e0 = (a0 + (i // 8) * 128) + (
                    lax.shift_right_logical(g, 7) * 1024 + (g & 127))
                sl = i * L
                ent_v[pl.ds(sl, L)] = e0
                ent_v[pl.ds(UW + sl, L)] = e0 + PLANE
                ent_v[pl.ds(2 * UW + sl, L)] = e0 + 2 * PLANE
                return carry

            lax.fori_loop(0, UW // L, build, 0)

            pltpu.async_copy(xyz_hbm.at[ent_v], words_v, sem).wait()

            obase = b_hi * (BL * M) + m_hi * (BL * ML)
            for c in range(3):
                pltpu.sync_copy(
                    words_v.at[pl.ds(c * UW, UW)],
                    out_hbm.at[pl.ds(c * OPLANE + obase, UW)])

    return sc_gather


_sc_gather = _make_sc_gather()


def kernel(batch_sample_xyz, input):
    xyz_flat = (
        batch_sample_xyz.transpose(2, 0, 1)
        .reshape(3, BH, BL, NH, 128)
        .transpose(0, 1, 3, 2, 4)
        .reshape(3 * B * N))
    idx_flat = (
        input.astype(jnp.int32)
        .reshape(BH, BL, MH, ML)
        .transpose(0, 2, 1, 3)
        .reshape(B * M))
    out1d = _sc_gather(xyz_flat, idx_flat)
    out = (
        out1d.reshape(3, BH, MH, BL, ML)
        .transpose(1, 3, 2, 4, 0)
        .reshape(B, M, 3))
    return out


# speedup vs baseline: 119.9713x; 1.0421x over previous
import functools

import jax
import jax.numpy as jnp
from jax import lax
from jax.experimental import pallas as pl
from jax.experimental.pallas import tpu as pltpu
from jax.experimental.pallas import tpu_sc as plsc

B, N, M = 16, 65536, 4096
NC, NS, L = 2, 16, 16
NW = NC * NS
BH, BL = B // 8, 8
MH, ML = M // 128, 128
NH = N // 128
U = BH * MH
UW = BL * ML
PLANE = B * N
OPLANE = B * M


def _make_sc_gather():
    mesh = plsc.VectorSubcoreMesh(core_axis_name="c", subcore_axis_name="s")

    TW = 2 * UW
    UNROLL = 8

    @functools.partial(
        pl.kernel,
        mesh=mesh,
        compiler_params=pltpu.CompilerParams(
            use_tc_tiling_on_sc=False, needs_layout_passes=False),
        out_type=jax.ShapeDtypeStruct((3 * B * M,), jnp.float32),
        scratch_types=[
            pltpu.VMEM((TW,), jnp.int32),
            pltpu.VMEM((3 * TW,), jnp.int32),
            pltpu.VMEM((3 * TW,), jnp.float32),
            pltpu.SemaphoreType.DMA,
        ],
    )
    def sc_gather(xyz_hbm, idx_hbm, out_hbm, idx_v, ent_v, words_v, sem):
        wid = lax.axis_index("s") * NC + lax.axis_index("c")
        u0 = wid * 2
        b_hi = u0 // MH
        a0 = b_hi * (BL * N)

        pltpu.sync_copy(idx_hbm.at[pl.ds(u0 * UW, TW)], idx_v)

        def build(q, carry):
            for t in range(UNROLL):
                i = q * UNROLL + t
                sl = i * L
                g = idx_v[pl.ds(sl, L)]
                b_lo = lax.rem(i, UW // L) // (ML // L)
                e0 = (a0 + b_lo * ML) + (
                    lax.shift_right_logical(g, 7) * (BL * ML) + (g & 127))
                ent_v[pl.ds(sl, L)] = e0
                ent_v[pl.ds(TW + sl, L)] = e0 + PLANE
                ent_v[pl.ds(2 * TW + sl, L)] = e0 + 2 * PLANE
            return carry

        lax.fori_loop(0, TW // L // UNROLL, build, 0)

        pltpu.async_copy(xyz_hbm.at[ent_v], words_v, sem).wait()

        obase = b_hi * (BL * M) + (u0 % MH) * UW
        for c in range(3):
            pltpu.sync_copy(
                words_v.at[pl.ds(c * TW, TW)],
                out_hbm.at[pl.ds(c * OPLANE + obase, TW)])

    return sc_gather


_sc_gather = _make_sc_gather()


def kernel(batch_sample_xyz, input):
    xyz_flat = (
        batch_sample_xyz.transpose(2, 0, 1)
        .reshape(3, BH, BL, NH, 128)
        .transpose(0, 1, 3, 2, 4)
        .reshape(3 * B * N))
    idx_flat = (
        input.astype(jnp.int32)
        .reshape(BH, BL, MH, ML)
        .transpose(0, 2, 1, 3)
        .reshape(B * M))
    out1d = _sc_gather(xyz_flat, idx_flat)
    out = (
        out1d.reshape(3, BH, MH, BL, ML)
        .transpose(1, 3, 2, 4, 0)
        .reshape(B, M, 3))
    return out


# speedup vs baseline: 124.4481x; 1.0373x over previous
import functools

import jax
import jax.numpy as jnp
from jax import lax
from jax.experimental import pallas as pl
from jax.experimental.pallas import tpu as pltpu
from jax.experimental.pallas import tpu_sc as plsc

B, N, M = 16, 65536, 4096
NC, NS, L = 2, 16, 16
NW = NC * NS
BH, BL = B // 8, 8
MH, ML = M // 128, 128
NH = N // 128
U = BH * MH
UW = BL * ML
PLANE = B * N
OPLANE = B * M


def _make_sc_gather():
    mesh = plsc.VectorSubcoreMesh(core_axis_name="c", subcore_axis_name="s")

    TW = 2 * UW
    UNROLL = 8

    @functools.partial(
        pl.kernel,
        mesh=mesh,
        compiler_params=pltpu.CompilerParams(
            use_tc_tiling_on_sc=False, needs_layout_passes=False),
        out_type=jax.ShapeDtypeStruct((3 * B * M,), jnp.float32),
        scratch_types=[
            pltpu.VMEM((TW,), jnp.int32),
            pltpu.VMEM((3 * TW,), jnp.int32),
            pltpu.VMEM((3 * TW,), jnp.float32),
            pltpu.SemaphoreType.DMA,
            pltpu.SemaphoreType.DMA,
            pltpu.SemaphoreType.DMA,
            pltpu.SemaphoreType.DMA,
        ],
    )
    def sc_gather(xyz_hbm, idx_hbm, out_hbm, idx_v, ent_v, words_v,
                  sem_i0, sem_i1, sem_g0, sem_g1):
        wid = lax.axis_index("s") * NC + lax.axis_index("c")
        u0 = wid * 2
        b_hi = u0 // MH
        a0 = b_hi * (BL * N)
        obase = b_hi * (BL * M) + (u0 % MH) * UW
        sem_i = (sem_i0, sem_i1)
        sem_g = (sem_g0, sem_g1)

        cp_i = [
            pltpu.async_copy(
                idx_hbm.at[pl.ds((u0 + h) * UW, UW)],
                idx_v.at[pl.ds(h * UW, UW)], sem_i[h])
            for h in range(2)
        ]

        cp_g = []
        for h in range(2):
            cp_i[h].wait()

            def build(q, carry, h=h):
                for t in range(UNROLL):
                    i = q * UNROLL + t
                    sl = h * UW + i * L
                    esl = h * 3 * UW + i * L
                    g = idx_v[pl.ds(sl, L)]
                    e0 = (a0 + (i // (ML // L)) * ML) + (
                        lax.shift_right_logical(g, 7) * (BL * ML) + (g & 127))
                    ent_v[pl.ds(esl, L)] = e0
                    ent_v[pl.ds(UW + esl, L)] = e0 + PLANE
                    ent_v[pl.ds(2 * UW + esl, L)] = e0 + 2 * PLANE
                return carry

            lax.fori_loop(0, UW // L // UNROLL, build, 0)

            cp_g.append(pltpu.async_copy(
                xyz_hbm.at[ent_v.at[pl.ds(h * 3 * UW, 3 * UW)]],
                words_v.at[pl.ds(h * 3 * UW, 3 * UW)], sem_g[h]))

        for h in range(2):
            cp_g[h].wait()
            for c in range(3):
                pltpu.sync_copy(
                    words_v.at[pl.ds(h * 3 * UW + c * UW, UW)],
                    out_hbm.at[pl.ds(c * OPLANE + obase + h * UW, UW)])

    return sc_gather


_sc_gather = _make_sc_gather()


def kernel(batch_sample_xyz, input):
    xyz_flat = (
        batch_sample_xyz.transpose(2, 0, 1)
        .reshape(3, BH, BL, NH, 128)
        .transpose(0, 1, 3, 2, 4)
        .reshape(3 * B * N))
    idx_flat = (
        input.astype(jnp.int32)
        .reshape(BH, BL, MH, ML)
        .transpose(0, 2, 1, 3)
        .reshape(B * M))
    out1d = _sc_gather(xyz_flat, idx_flat)
    out = (
        out1d.reshape(3, BH, MH, BL, ML)
        .transpose(1, 3, 2, 4, 0)
        .reshape(B, M, 3))
    return out
